# baseline (speedup 1.0000x reference)
import dataclasses
import functools

import jax
import jax.numpy as jnp
from jax import lax
from jax.experimental import pallas as pl
from jax.experimental.pallas import tpu as pltpu
from jax.experimental.pallas import tpu_sc as plsc

B = 16384
D = 128
T_ROWS = 352
NC = 2
NS = 16
NW = NC * NS
BPW = B // NW
NGRP = BPW // 16
HI = lax.Precision.HIGHEST
F32 = jnp.float32
BF16 = jnp.bfloat16


def _prep_body(xt_ref, aug_ref, mag_ref, cls_ref, w0_ref, t_ref, idx_ref):
    pa = jnp.dot(aug_ref[...], w0_ref[0:128, :], precision=HI)
    pm = jnp.dot(mag_ref[...], w0_ref[128:256, :], precision=HI)
    pc = jnp.dot(cls_ref[...], w0_ref[256:384, :], precision=HI)
    ra = lax.broadcasted_iota(jnp.int32, (250, 25), 0) // 10
    ca = lax.broadcasted_iota(jnp.int32, (250, 25), 1)
    ea = (ra == ca).astype(F32)
    rm = lax.broadcasted_iota(jnp.int32, (250, 10), 0) % 10
    cm = lax.broadcasted_iota(jnp.int32, (250, 10), 1)
    em = (rm == cm).astype(F32)
    pair = 0.5 * (jnp.dot(ea, pa, precision=HI) + jnp.dot(em, pm, precision=HI))
    t_ref[...] = jnp.concatenate([pair, pc, jnp.zeros((2, D), F32)], axis=0)
    x0 = xt_ref[0:1, :]
    x1 = xt_ref[1:2, :]
    x2 = xt_ref[2:3, :]
    x3 = xt_ref[3:4, :]
    x4 = xt_ref[4:5, :]
    ia = (10 * jnp.clip(x0, 0, 24) + jnp.clip(x1, 0, 9)) * D
    ib = (10 * jnp.clip(x2, 0, 24) + jnp.clip(x3, 0, 9)) * D
    ic = (250 + jnp.clip(x4, 0, 99)) * D
    idx_ref[...] = jnp.concatenate(
        [ia, ib, ic, jnp.zeros((5, B), jnp.int32)], axis=0)


def _sc_gather_sum(t_hbm, idx_hbm, out_hbm,
                   t_loc, obuf, ia_v, ib_v, ic_v, s0, s1, s2, s3):
    cid = lax.axis_index("c")
    sid = lax.axis_index("s")
    wid = cid * NS + sid
    base = wid * BPW
    d0 = pltpu.async_copy(t_hbm, t_loc, s0)
    d1 = pltpu.async_copy(idx_hbm.at[0, pl.ds(base, BPW)], ia_v, s1)
    d2 = pltpu.async_copy(idx_hbm.at[1, pl.ds(base, BPW)], ib_v, s2)
    d3 = pltpu.async_copy(idx_hbm.at[2, pl.ds(base, BPW)], ic_v, s3)
    d0.wait()
    d1.wait()
    d2.wait()
    d3.wait()

    @pl.loop(0, NGRP)
    def _(g):
        s16 = pl.ds(g * 16, 16)
        a0 = ia_v.at[s16][...]
        b0 = ib_v.at[s16][...]
        c0 = ic_v.at[s16][...]
        for c in range(D):
            va = plsc.load_gather(t_loc, [a0 + c])
            vb = plsc.load_gather(t_loc, [b0 + c])
            vc = plsc.load_gather(t_loc, [c0 + c])
            obuf.at[c, s16][...] = va + vb + vc

    pltpu.sync_copy(obuf, out_hbm.at[wid])


def _mm3t(w, h):
    dn = (((0,), (0,)), ((), ()))
    w_hi = w.astype(BF16)
    w_lo = (w - w_hi.astype(F32)).astype(BF16)
    h_hi = h.astype(BF16)
    h_lo = (h - h_hi.astype(F32)).astype(BF16)
    return (lax.dot_general(w_hi, h_hi, dn, preferred_element_type=F32)
            + (lax.dot_general(w_lo, h_hi, dn, preferred_element_type=F32)
               + lax.dot_general(w_hi, h_lo, dn, preferred_element_type=F32)))


def _mlp_body(pre_ref, b0_ref, w1_ref, b1_ref, w2_ref, b2_ref, wout_ref,
              bout_ref, y_ref):
    h = jnp.maximum(pre_ref[0] + b0_ref[...], 0.0)
    h = jnp.maximum(_mm3t(w1_ref[...], h) + b1_ref[...], 0.0)
    h = jnp.maximum(_mm3t(w2_ref[...], h) + b2_ref[...], 0.0)
    y_ref[0] = _mm3t(wout_ref[...], h) + bout_ref[...]


@jax.jit
def kernel(x, aug_table, mag_table, cls_table, W0, b0, W1, b1, W2, b2, Wout,
           bout):
    xt = jnp.zeros((8, B), jnp.int32).at[0:5, :].set(x.T.astype(jnp.int32))

    t_tab, idx = pl.pallas_call(
        _prep_body,
        grid=(1,),
        in_specs=[
            pl.BlockSpec((8, B), lambda i: (0, 0)),
            pl.BlockSpec((25, D), lambda i: (0, 0)),
            pl.BlockSpec((10, D), lambda i: (0, 0)),
            pl.BlockSpec((100, D), lambda i: (0, 0)),
            pl.BlockSpec((384, D), lambda i: (0, 0)),
        ],
        out_specs=[
            pl.BlockSpec((T_ROWS, D), lambda i: (0, 0)),
            pl.BlockSpec((8, B), lambda i: (0, 0)),
        ],
        out_shape=[
            jax.ShapeDtypeStruct((T_ROWS, D), F32),
            jax.ShapeDtypeStruct((8, B), jnp.int32),
        ],
    )(xt, aug_table, mag_table, cls_table, W0)
    t_flat = t_tab.reshape(T_ROWS * D)

    sc_params = pltpu.CompilerParams()
    if "needs_layout_passes" in pltpu.CompilerParams.__dataclass_fields__:
        sc_params = dataclasses.replace(sc_params, needs_layout_passes=False)
    sc_fn = functools.partial(
        pl.kernel,
        out_type=jax.ShapeDtypeStruct((NW, D, BPW), F32),
        compiler_params=sc_params,
        mesh=plsc.VectorSubcoreMesh(core_axis_name="c", subcore_axis_name="s"),
        scratch_types=[
            pltpu.VMEM((T_ROWS * D,), F32),
            pltpu.VMEM((D, BPW), F32),
            pltpu.VMEM((BPW,), jnp.int32),
            pltpu.VMEM((BPW,), jnp.int32),
            pltpu.VMEM((BPW,), jnp.int32),
            pltpu.SemaphoreType.DMA,
            pltpu.SemaphoreType.DMA,
            pltpu.SemaphoreType.DMA,
            pltpu.SemaphoreType.DMA,
        ],
    )(_sc_gather_sum)
    pre3 = sc_fn(t_flat, idx)

    y3 = pl.pallas_call(
        _mlp_body,
        grid=(NW,),
        in_specs=[
            pl.BlockSpec((1, D, BPW), lambda i: (i, 0, 0)),
            pl.BlockSpec((D, 1), lambda i: (0, 0)),
            pl.BlockSpec((D, D), lambda i: (0, 0)),
            pl.BlockSpec((D, 1), lambda i: (0, 0)),
            pl.BlockSpec((D, D), lambda i: (0, 0)),
            pl.BlockSpec((D, 1), lambda i: (0, 0)),
            pl.BlockSpec((D, 1), lambda i: (0, 0)),
            pl.BlockSpec((1, 1), lambda i: (0, 0)),
        ],
        out_specs=pl.BlockSpec((1, 1, BPW), lambda i: (i, 0, 0)),
        out_shape=jax.ShapeDtypeStruct((NW, 1, BPW), F32),
    )(pre3, b0.reshape(D, 1), W1, b1.reshape(D, 1), W2, b2.reshape(D, 1),
      Wout, bout.reshape(1, 1))
    return y3.reshape(B, 1)


# speedup vs baseline: 3.0134x; 3.0134x over previous
import dataclasses
import functools

import jax
import jax.numpy as jnp
from jax import lax
from jax.experimental import pallas as pl
from jax.experimental.pallas import tpu as pltpu
from jax.experimental.pallas import tpu_sc as plsc

B = 16384
D = 128
T_ROWS = 352
NC = 2
NS = 16
NW = NC * NS
BPW = B // NW
NGRP = BPW // 16
HI = lax.Precision.HIGHEST
F32 = jnp.float32
BF16 = jnp.bfloat16


def _prep_body(xt_ref, aug_ref, mag_ref, cls_ref, w0_ref, t_ref, idx_ref):
    pa = jnp.dot(aug_ref[...], w0_ref[0:128, :], precision=HI)
    pm = jnp.dot(mag_ref[...], w0_ref[128:256, :], precision=HI)
    pc = jnp.dot(cls_ref[...], w0_ref[256:384, :], precision=HI)
    ra = lax.broadcasted_iota(jnp.int32, (250, 25), 0) // 10
    ca = lax.broadcasted_iota(jnp.int32, (250, 25), 1)
    ea = (ra == ca).astype(F32)
    rm = lax.broadcasted_iota(jnp.int32, (250, 10), 0) % 10
    cm = lax.broadcasted_iota(jnp.int32, (250, 10), 1)
    em = (rm == cm).astype(F32)
    pair = 0.5 * (jnp.dot(ea, pa, precision=HI) + jnp.dot(em, pm, precision=HI))
    t_ref[...] = jnp.concatenate([pair, pc, jnp.zeros((2, D), F32)], axis=0)
    x0 = xt_ref[0:1, :]
    x1 = xt_ref[1:2, :]
    x2 = xt_ref[2:3, :]
    x3 = xt_ref[3:4, :]
    x4 = xt_ref[4:5, :]
    ia = 10 * jnp.clip(x0, 0, 24) + jnp.clip(x1, 0, 9)
    ib = 10 * jnp.clip(x2, 0, 24) + jnp.clip(x3, 0, 9)
    ic = 250 + jnp.clip(x4, 0, 99)
    idx_ref[...] = jnp.concatenate(
        [ia, ib, ic, jnp.zeros((5, B), jnp.int32)], axis=0)


CHUNK = 128
NCHUNK = BPW // CHUNK


def _sc_gather_sum(t_hbm, idx_hbm, out_hbm,
                   t_sh, ia_v, ib_v, ic_v,
                   ga0, gb0, gc0, ga1, gb1, gc1,
                   sa0, sb0, sc0, sa1, sb1, sc1, so0, so1):
    cid = lax.axis_index("c")
    sid = lax.axis_index("s")
    wid = cid * NS + sid
    base = wid * BPW
    bufs = ((ga0, gb0, gc0, sa0, sb0, sc0, so0),
            (ga1, gb1, gc1, sa1, sb1, sc1, so1))

    @pl.when(sid == 0)
    def _():
        pltpu.sync_copy(t_hbm, t_sh)

    da = pltpu.async_copy(idx_hbm.at[0, pl.ds(base, BPW)], ia_v, sa0)
    db = pltpu.async_copy(idx_hbm.at[1, pl.ds(base, BPW)], ib_v, sb0)
    dc = pltpu.async_copy(idx_hbm.at[2, pl.ds(base, BPW)], ic_v, sc0)
    da.wait()
    db.wait()
    dc.wait()
    plsc.subcore_barrier()

    def issue_gathers(c):
        ga, gb, gc, sa, sb, sc, _ = bufs[c % 2]
        off = c * CHUNK
        return (pltpu.async_copy(t_sh.at[ia_v.at[pl.ds(off, CHUNK)]], ga, sa),
                pltpu.async_copy(t_sh.at[ib_v.at[pl.ds(off, CHUNK)]], gb, sb),
                pltpu.async_copy(t_sh.at[ic_v.at[pl.ds(off, CHUNK)]], gc, sc))

    pend_g = issue_gathers(0)
    pend_out = [None, None]
    for c in range(NCHUNK):
        ga, gb, gc, _, _, _, so = bufs[c % 2]
        for d in pend_g:
            d.wait()
        if c + 1 < NCHUNK:
            prev_out = pend_out[(c + 1) % 2]
            if prev_out is not None:
                prev_out.wait()
            pend_g = issue_gathers(c + 1)

        @pl.loop(0, CHUNK)
        def _(r):
            for g8 in range(8):
                slc = (r, pl.ds(g8 * 16, 16))
                ga.at[slc][...] = (
                    ga.at[slc][...] + gb.at[slc][...] + gc.at[slc][...])

        pend_out[c % 2] = pltpu.async_copy(
            ga, out_hbm.at[pl.ds(base + c * CHUNK, CHUNK)], so)
    for d in pend_out:
        if d is not None:
            d.wait()


def _mm3(a, b):
    a_hi = a.astype(BF16)
    a_lo = (a - a_hi.astype(F32)).astype(BF16)
    b_hi = b.astype(BF16)
    b_lo = (b - b_hi.astype(F32)).astype(BF16)
    return (jnp.dot(a_hi, b_hi, preferred_element_type=F32)
            + (jnp.dot(a_hi, b_lo, preferred_element_type=F32)
               + jnp.dot(a_lo, b_hi, preferred_element_type=F32)))


def _mlp_body(pre_ref, b0_ref, w1_ref, b1_ref, w2_ref, b2_ref, wout_ref,
              bout_ref, y_ref):
    h = jnp.maximum(pre_ref[...] + b0_ref[...], 0.0)
    h = jnp.maximum(_mm3(h, w1_ref[...]) + b1_ref[...], 0.0)
    h = jnp.maximum(_mm3(h, w2_ref[...]) + b2_ref[...], 0.0)
    y_ref[...] = _mm3(h, wout_ref[...]) + bout_ref[...]


@jax.jit
def kernel(x, aug_table, mag_table, cls_table, W0, b0, W1, b1, W2, b2, Wout,
           bout):
    xt = jnp.zeros((8, B), jnp.int32).at[0:5, :].set(x.T.astype(jnp.int32))

    t_tab, idx = pl.pallas_call(
        _prep_body,
        grid=(1,),
        in_specs=[
            pl.BlockSpec((8, B), lambda i: (0, 0)),
            pl.BlockSpec((25, D), lambda i: (0, 0)),
            pl.BlockSpec((10, D), lambda i: (0, 0)),
            pl.BlockSpec((100, D), lambda i: (0, 0)),
            pl.BlockSpec((384, D), lambda i: (0, 0)),
        ],
        out_specs=[
            pl.BlockSpec((T_ROWS, D), lambda i: (0, 0)),
            pl.BlockSpec((8, B), lambda i: (0, 0)),
        ],
        out_shape=[
            jax.ShapeDtypeStruct((T_ROWS, D), F32),
            jax.ShapeDtypeStruct((8, B), jnp.int32),
        ],
    )(xt, aug_table, mag_table, cls_table, W0)

    sc_params = pltpu.CompilerParams()
    if "needs_layout_passes" in pltpu.CompilerParams.__dataclass_fields__:
        sc_params = dataclasses.replace(sc_params, needs_layout_passes=False)
    sc_fn = functools.partial(
        pl.kernel,
        out_type=jax.ShapeDtypeStruct((B, D), F32),
        compiler_params=sc_params,
        mesh=plsc.VectorSubcoreMesh(core_axis_name="c", subcore_axis_name="s"),
        scratch_types=[
            pltpu.VMEM_SHARED((T_ROWS, D), F32),
            pltpu.VMEM((BPW,), jnp.int32),
            pltpu.VMEM((BPW,), jnp.int32),
            pltpu.VMEM((BPW,), jnp.int32),
            pltpu.VMEM((CHUNK, D), F32),
            pltpu.VMEM((CHUNK, D), F32),
            pltpu.VMEM((CHUNK, D), F32),
            pltpu.VMEM((CHUNK, D), F32),
            pltpu.VMEM((CHUNK, D), F32),
            pltpu.VMEM((CHUNK, D), F32),
            pltpu.SemaphoreType.DMA,
            pltpu.SemaphoreType.DMA,
            pltpu.SemaphoreType.DMA,
            pltpu.SemaphoreType.DMA,
            pltpu.SemaphoreType.DMA,
            pltpu.SemaphoreType.DMA,
            pltpu.SemaphoreType.DMA,
            pltpu.SemaphoreType.DMA,
        ],
    )(_sc_gather_sum)
    pre = sc_fn(t_tab, idx)

    y = pl.pallas_call(
        _mlp_body,
        grid=(B // 1024,),
        in_specs=[
            pl.BlockSpec((1024, D), lambda i: (i, 0)),
            pl.BlockSpec((1, D), lambda i: (0, 0)),
            pl.BlockSpec((D, D), lambda i: (0, 0)),
            pl.BlockSpec((1, D), lambda i: (0, 0)),
            pl.BlockSpec((D, D), lambda i: (0, 0)),
            pl.BlockSpec((1, D), lambda i: (0, 0)),
            pl.BlockSpec((D, 1), lambda i: (0, 0)),
            pl.BlockSpec((1, 1), lambda i: (0, 0)),
        ],
        out_specs=pl.BlockSpec((1024, 1), lambda i: (i, 0)),
        out_shape=jax.ShapeDtypeStruct((B, 1), F32),
    )(pre, b0.reshape(1, D), W1, b1.reshape(1, D), W2, b2.reshape(1, D),
      Wout, bout.reshape(1, 1))
    return y


# speedup vs baseline: 3.1532x; 1.0464x over previous
import dataclasses
import functools

import jax
import jax.numpy as jnp
from jax import lax
from jax.experimental import pallas as pl
from jax.experimental.pallas import tpu as pltpu
from jax.experimental.pallas import tpu_sc as plsc

B = 16384
D = 128
T_ROWS = 352
NC = 2
NS = 16
NW = NC * NS
BPW = B // NW
NGRP = BPW // 16
HI = lax.Precision.HIGHEST
F32 = jnp.float32
BF16 = jnp.bfloat16


def _prep_body(xt_ref, aug_ref, mag_ref, cls_ref, w0_ref, t_ref, idx_ref):
    pa = jnp.dot(aug_ref[...], w0_ref[0:128, :], precision=HI)
    pm = jnp.dot(mag_ref[...], w0_ref[128:256, :], precision=HI)
    pc = jnp.dot(cls_ref[...], w0_ref[256:384, :], precision=HI)
    ra = lax.broadcasted_iota(jnp.int32, (250, 25), 0) // 10
    ca = lax.broadcasted_iota(jnp.int32, (250, 25), 1)
    ea = (ra == ca).astype(F32)
    rm = lax.broadcasted_iota(jnp.int32, (250, 10), 0) % 10
    cm = lax.broadcasted_iota(jnp.int32, (250, 10), 1)
    em = (rm == cm).astype(F32)
    pair = 0.5 * (jnp.dot(ea, pa, precision=HI) + jnp.dot(em, pm, precision=HI))
    t_ref[...] = jnp.concatenate([pair, pc, jnp.zeros((2, D), F32)], axis=0)
    x0 = xt_ref[0:1, :]
    x1 = xt_ref[1:2, :]
    x2 = xt_ref[2:3, :]
    x3 = xt_ref[3:4, :]
    x4 = xt_ref[4:5, :]
    ia = 10 * jnp.clip(x0, 0, 24) + jnp.clip(x1, 0, 9)
    ib = 10 * jnp.clip(x2, 0, 24) + jnp.clip(x3, 0, 9)
    ic = 250 + jnp.clip(x4, 0, 99)
    idx_ref[...] = jnp.concatenate(
        [ia, ib, ic, jnp.zeros((5, B), jnp.int32)], axis=0)


CHUNK = 128
NCHUNK = BPW // CHUNK


def _sc_gather_sum(t_hbm, idx_hbm, out_hbm,
                   t_sh, ia_v, ib_v, ic_v,
                   ga0, gb0, gc0, ga1, gb1, gc1,
                   sa0, sb0, sc0, sa1, sb1, sc1, so0, so1):
    cid = lax.axis_index("c")
    sid = lax.axis_index("s")
    wid = cid * NS + sid
    base = wid * BPW
    bufs = ((ga0, gb0, gc0, sa0, sb0, sc0, so0),
            (ga1, gb1, gc1, sa1, sb1, sc1, so1))

    @pl.when(sid == 0)
    def _():
        pltpu.sync_copy(t_hbm, t_sh)

    da = pltpu.async_copy(idx_hbm.at[0, pl.ds(base, BPW)], ia_v, sa0)
    db = pltpu.async_copy(idx_hbm.at[1, pl.ds(base, BPW)], ib_v, sb0)
    dc = pltpu.async_copy(idx_hbm.at[2, pl.ds(base, BPW)], ic_v, sc0)
    da.wait()
    db.wait()
    dc.wait()
    plsc.subcore_barrier()

    def issue_gathers(c):
        ga, gb, gc, sa, sb, sc, _ = bufs[c % 2]
        off = c * CHUNK
        return (pltpu.async_copy(t_sh.at[ia_v.at[pl.ds(off, CHUNK)]], ga, sa),
                pltpu.async_copy(t_sh.at[ib_v.at[pl.ds(off, CHUNK)]], gb, sb),
                pltpu.async_copy(t_sh.at[ic_v.at[pl.ds(off, CHUNK)]], gc, sc))

    pend_g = issue_gathers(0)
    pend_out = [None, None]
    for c in range(NCHUNK):
        ga, gb, gc, _, _, _, so = bufs[c % 2]
        for d in pend_g:
            d.wait()
        if c + 1 < NCHUNK:
            prev_out = pend_out[(c + 1) % 2]
            if prev_out is not None:
                prev_out.wait()
            pend_g = issue_gathers(c + 1)

        @pl.loop(0, CHUNK)
        def _(r):
            for g8 in range(8):
                slc = (r, pl.ds(g8 * 16, 16))
                ga.at[slc][...] = (
                    ga.at[slc][...] + gb.at[slc][...] + gc.at[slc][...])

        pend_out[c % 2] = pltpu.async_copy(
            ga, out_hbm.at[pl.ds(base + c * CHUNK, CHUNK)], so)
    for d in pend_out:
        if d is not None:
            d.wait()


def _mm3(a, b_hi, b_lo):
    a_hi = a.astype(BF16)
    a_lo = (a - a_hi.astype(F32)).astype(BF16)
    return (jnp.dot(a_hi, b_hi, preferred_element_type=F32)
            + (jnp.dot(a_hi, b_lo, preferred_element_type=F32)
               + jnp.dot(a_lo, b_hi, preferred_element_type=F32)))


def _mlp_body(pre_ref, b0_ref, w1h_ref, w1l_ref, b1_ref, w2h_ref, w2l_ref,
              b2_ref, wouth_ref, woutl_ref, bout_ref, y_ref):
    h = jnp.maximum(pre_ref[...] + b0_ref[...], 0.0)
    h = jnp.maximum(_mm3(h, w1h_ref[...], w1l_ref[...]) + b1_ref[...], 0.0)
    h = jnp.maximum(_mm3(h, w2h_ref[...], w2l_ref[...]) + b2_ref[...], 0.0)
    y_ref[...] = _mm3(h, wouth_ref[...], woutl_ref[...]) + bout_ref[...]


@jax.jit
def kernel(x, aug_table, mag_table, cls_table, W0, b0, W1, b1, W2, b2, Wout,
           bout):
    xt = jnp.zeros((8, B), jnp.int32).at[0:5, :].set(x.T.astype(jnp.int32))

    t_tab, idx = pl.pallas_call(
        _prep_body,
        grid=(1,),
        in_specs=[
            pl.BlockSpec((8, B), lambda i: (0, 0)),
            pl.BlockSpec((25, D), lambda i: (0, 0)),
            pl.BlockSpec((10, D), lambda i: (0, 0)),
            pl.BlockSpec((100, D), lambda i: (0, 0)),
            pl.BlockSpec((384, D), lambda i: (0, 0)),
        ],
        out_specs=[
            pl.BlockSpec((T_ROWS, D), lambda i: (0, 0)),
            pl.BlockSpec((8, B), lambda i: (0, 0)),
        ],
        out_shape=[
            jax.ShapeDtypeStruct((T_ROWS, D), F32),
            jax.ShapeDtypeStruct((8, B), jnp.int32),
        ],
    )(xt, aug_table, mag_table, cls_table, W0)

    sc_params = pltpu.CompilerParams()
    if "needs_layout_passes" in pltpu.CompilerParams.__dataclass_fields__:
        sc_params = dataclasses.replace(sc_params, needs_layout_passes=False)
    sc_fn = functools.partial(
        pl.kernel,
        out_type=jax.ShapeDtypeStruct((B, D), F32),
        compiler_params=sc_params,
        mesh=plsc.VectorSubcoreMesh(core_axis_name="c", subcore_axis_name="s"),
        scratch_types=[
            pltpu.VMEM_SHARED((T_ROWS, D), F32),
            pltpu.VMEM((BPW,), jnp.int32),
            pltpu.VMEM((BPW,), jnp.int32),
            pltpu.VMEM((BPW,), jnp.int32),
            pltpu.VMEM((CHUNK, D), F32),
            pltpu.VMEM((CHUNK, D), F32),
            pltpu.VMEM((CHUNK, D), F32),
            pltpu.VMEM((CHUNK, D), F32),
            pltpu.VMEM((CHUNK, D), F32),
            pltpu.VMEM((CHUNK, D), F32),
            pltpu.SemaphoreType.DMA,
            pltpu.SemaphoreType.DMA,
            pltpu.SemaphoreType.DMA,
            pltpu.SemaphoreType.DMA,
            pltpu.SemaphoreType.DMA,
            pltpu.SemaphoreType.DMA,
            pltpu.SemaphoreType.DMA,
            pltpu.SemaphoreType.DMA,
        ],
    )(_sc_gather_sum)
    pre = sc_fn(t_tab, idx)

    def split(w):
        w_hi = w.astype(BF16)
        return w_hi, (w - w_hi.astype(F32)).astype(BF16)

    w1h, w1l = split(W1)
    w2h, w2l = split(W2)
    wouth, woutl = split(Wout)
    y = pl.pallas_call(
        _mlp_body,
        grid=(B // 2048,),
        in_specs=[
            pl.BlockSpec((2048, D), lambda i: (i, 0)),
            pl.BlockSpec((1, D), lambda i: (0, 0)),
            pl.BlockSpec((D, D), lambda i: (0, 0)),
            pl.BlockSpec((D, D), lambda i: (0, 0)),
            pl.BlockSpec((1, D), lambda i: (0, 0)),
            pl.BlockSpec((D, D), lambda i: (0, 0)),
            pl.BlockSpec((D, D), lambda i: (0, 0)),
            pl.BlockSpec((1, D), lambda i: (0, 0)),
            pl.BlockSpec((D, 1), lambda i: (0, 0)),
            pl.BlockSpec((D, 1), lambda i: (0, 0)),
            pl.BlockSpec((1, 1), lambda i: (0, 0)),
        ],
        out_specs=pl.BlockSpec((2048, 1), lambda i: (i, 0)),
        out_shape=jax.ShapeDtypeStruct((B, 1), F32),
    )(pre, b0.reshape(1, D), w1h, w1l, b1.reshape(1, D), w2h, w2l,
      b2.reshape(1, D), wouth, woutl, bout.reshape(1, 1))
    return y


# speedup vs baseline: 3.3082x; 1.0492x over previous
import dataclasses
import functools

import jax
import jax.numpy as jnp
from jax import lax
from jax.experimental import pallas as pl
from jax.experimental.pallas import tpu as pltpu
from jax.experimental.pallas import tpu_sc as plsc

B = 16384
D = 128
T_ROWS = 352
NC = 2
NS = 16
NW = NC * NS
BPW = B // NW
NGRP = BPW // 16
HI = lax.Precision.HIGHEST
F32 = jnp.float32
BF16 = jnp.bfloat16


def _prep_body(xt_ref, aug_ref, mag_ref, cls_ref, w0_ref, t_ref, idx_ref):
    pa = jnp.dot(aug_ref[...], w0_ref[0:128, :], precision=HI)
    pm = jnp.dot(mag_ref[...], w0_ref[128:256, :], precision=HI)
    pc = jnp.dot(cls_ref[...], w0_ref[256:384, :], precision=HI)
    ra = lax.broadcasted_iota(jnp.int32, (250, 25), 0) // 10
    ca = lax.broadcasted_iota(jnp.int32, (250, 25), 1)
    ea = (ra == ca).astype(F32)
    rm = lax.broadcasted_iota(jnp.int32, (250, 10), 0) % 10
    cm = lax.broadcasted_iota(jnp.int32, (250, 10), 1)
    em = (rm == cm).astype(F32)
    pair = 0.5 * (jnp.dot(ea, pa, precision=HI) + jnp.dot(em, pm, precision=HI))
    t_ref[...] = jnp.concatenate([pair, pc, jnp.zeros((2, D), F32)], axis=0)
    x0 = xt_ref[0:1, :]
    x1 = xt_ref[1:2, :]
    x2 = xt_ref[2:3, :]
    x3 = xt_ref[3:4, :]
    x4 = xt_ref[4:5, :]
    ia = 10 * jnp.clip(x0, 0, 24) + jnp.clip(x1, 0, 9)
    ib = 10 * jnp.clip(x2, 0, 24) + jnp.clip(x3, 0, 9)
    ic = 250 + jnp.clip(x4, 0, 99)
    idx_ref[...] = jnp.concatenate(
        [ia, ib, ic, jnp.zeros((5, B), jnp.int32)], axis=0)


CHUNK = 128
HALF = B // 2
BPW_H = HALF // NW
NCHUNK = BPW_H // CHUNK


def _sc_gather_sum(half, t_hbm, idx_hbm, out_hbm,
                   t_sh, ia_v, ib_v, ic_v,
                   ga0, gb0, gc0, ga1, gb1, gc1,
                   sa0, sb0, sc0, sa1, sb1, sc1, so0, so1):
    cid = lax.axis_index("c")
    sid = lax.axis_index("s")
    wid = cid * NS + sid
    base = half * HALF + wid * BPW_H
    bufs = ((ga0, gb0, gc0, sa0, sb0, sc0, so0),
            (ga1, gb1, gc1, sa1, sb1, sc1, so1))

    @pl.when(sid == 0)
    def _():
        pltpu.sync_copy(t_hbm, t_sh)

    da = pltpu.async_copy(idx_hbm.at[0, pl.ds(base, BPW_H)], ia_v, sa0)
    db = pltpu.async_copy(idx_hbm.at[1, pl.ds(base, BPW_H)], ib_v, sb0)
    dc = pltpu.async_copy(idx_hbm.at[2, pl.ds(base, BPW_H)], ic_v, sc0)
    da.wait()
    db.wait()
    dc.wait()
    plsc.subcore_barrier()

    def issue_gathers(c):
        ga, gb, gc, sa, sb, sc, _ = bufs[c % 2]
        off = c * CHUNK
        return (pltpu.async_copy(t_sh.at[ia_v.at[pl.ds(off, CHUNK)]], ga, sa),
                pltpu.async_copy(t_sh.at[ib_v.at[pl.ds(off, CHUNK)]], gb, sb),
                pltpu.async_copy(t_sh.at[ic_v.at[pl.ds(off, CHUNK)]], gc, sc))

    pend_g = issue_gathers(0)
    pend_out = [None, None]
    for c in range(NCHUNK):
        ga, gb, gc, _, _, _, so = bufs[c % 2]
        for d in pend_g:
            d.wait()
        if c + 1 < NCHUNK:
            prev_out = pend_out[(c + 1) % 2]
            if prev_out is not None:
                prev_out.wait()
            pend_g = issue_gathers(c + 1)

        @pl.loop(0, CHUNK)
        def _(r):
            for g8 in range(8):
                slc = (r, pl.ds(g8 * 16, 16))
                ga.at[slc][...] = (
                    ga.at[slc][...] + gb.at[slc][...] + gc.at[slc][...])

        pend_out[c % 2] = pltpu.async_copy(
            ga, out_hbm.at[pl.ds(wid * BPW_H + c * CHUNK, CHUNK)], so)
    for d in pend_out:
        if d is not None:
            d.wait()


def _mm3(a, b_hi, b_lo):
    a_hi = a.astype(BF16)
    a_lo = (a - a_hi.astype(F32)).astype(BF16)
    return (jnp.dot(a_hi, b_hi, preferred_element_type=F32)
            + (jnp.dot(a_hi, b_lo, preferred_element_type=F32)
               + jnp.dot(a_lo, b_hi, preferred_element_type=F32)))


def _mlp_body(pre_ref, b0_ref, w1h_ref, w1l_ref, b1_ref, w2h_ref, w2l_ref,
              b2_ref, wouth_ref, woutl_ref, bout_ref, y_ref):
    h = jnp.maximum(pre_ref[...] + b0_ref[...], 0.0)
    h = jnp.maximum(_mm3(h, w1h_ref[...], w1l_ref[...]) + b1_ref[...], 0.0)
    h = jnp.maximum(_mm3(h, w2h_ref[...], w2l_ref[...]) + b2_ref[...], 0.0)
    y_ref[...] = _mm3(h, wouth_ref[...], woutl_ref[...]) + bout_ref[...]


@jax.jit
def kernel(x, aug_table, mag_table, cls_table, W0, b0, W1, b1, W2, b2, Wout,
           bout):
    xt = x.T.astype(jnp.int32)

    t_tab, idx = pl.pallas_call(
        _prep_body,
        grid=(1,),
        in_specs=[
            pl.BlockSpec((5, B), lambda i: (0, 0)),
            pl.BlockSpec((25, D), lambda i: (0, 0)),
            pl.BlockSpec((10, D), lambda i: (0, 0)),
            pl.BlockSpec((100, D), lambda i: (0, 0)),
            pl.BlockSpec((384, D), lambda i: (0, 0)),
        ],
        out_specs=[
            pl.BlockSpec((T_ROWS, D), lambda i: (0, 0)),
            pl.BlockSpec((8, B), lambda i: (0, 0)),
        ],
        out_shape=[
            jax.ShapeDtypeStruct((T_ROWS, D), F32),
            jax.ShapeDtypeStruct((8, B), jnp.int32),
        ],
    )(xt, aug_table, mag_table, cls_table, W0)

    sc_params = pltpu.CompilerParams()
    if "needs_layout_passes" in pltpu.CompilerParams.__dataclass_fields__:
        sc_params = dataclasses.replace(sc_params, needs_layout_passes=False)
    sc_kernel = functools.partial(
        pl.kernel,
        out_type=jax.ShapeDtypeStruct((HALF, D), F32),
        compiler_params=sc_params,
        mesh=plsc.VectorSubcoreMesh(core_axis_name="c", subcore_axis_name="s"),
        scratch_types=[
            pltpu.VMEM_SHARED((T_ROWS, D), F32),
            pltpu.VMEM((BPW_H,), jnp.int32),
            pltpu.VMEM((BPW_H,), jnp.int32),
            pltpu.VMEM((BPW_H,), jnp.int32),
            pltpu.VMEM((CHUNK, D), F32),
            pltpu.VMEM((CHUNK, D), F32),
            pltpu.VMEM((CHUNK, D), F32),
            pltpu.VMEM((CHUNK, D), F32),
            pltpu.VMEM((CHUNK, D), F32),
            pltpu.VMEM((CHUNK, D), F32),
            pltpu.SemaphoreType.DMA,
            pltpu.SemaphoreType.DMA,
            pltpu.SemaphoreType.DMA,
            pltpu.SemaphoreType.DMA,
            pltpu.SemaphoreType.DMA,
            pltpu.SemaphoreType.DMA,
            pltpu.SemaphoreType.DMA,
            pltpu.SemaphoreType.DMA,
        ],
    )
    pre0 = sc_kernel(functools.partial(_sc_gather_sum, 0))(t_tab, idx)
    pre1 = sc_kernel(functools.partial(_sc_gather_sum, 1))(t_tab, idx)

    def split(w):
        w_hi = w.astype(BF16)
        return w_hi, (w - w_hi.astype(F32)).astype(BF16)

    w1h, w1l = split(W1)
    w2h, w2l = split(W2)
    wouth, woutl = split(Wout)
    def mlp(pre_h):
        return pl.pallas_call(
            _mlp_body,
            grid=(HALF // 2048,),
            in_specs=[
                pl.BlockSpec((2048, D), lambda i: (i, 0)),
                pl.BlockSpec((1, D), lambda i: (0, 0)),
                pl.BlockSpec((D, D), lambda i: (0, 0)),
                pl.BlockSpec((D, D), lambda i: (0, 0)),
                pl.BlockSpec((1, D), lambda i: (0, 0)),
                pl.BlockSpec((D, D), lambda i: (0, 0)),
                pl.BlockSpec((D, D), lambda i: (0, 0)),
                pl.BlockSpec((1, D), lambda i: (0, 0)),
                pl.BlockSpec((D, 1), lambda i: (0, 0)),
                pl.BlockSpec((D, 1), lambda i: (0, 0)),
                pl.BlockSpec((1, 1), lambda i: (0, 0)),
            ],
            out_specs=pl.BlockSpec((2048, 1), lambda i: (i, 0)),
            out_shape=jax.ShapeDtypeStruct((HALF, 1), F32),
        )(pre_h, b0.reshape(1, D), w1h, w1l, b1.reshape(1, D), w2h, w2l,
          b2.reshape(1, D), wouth, woutl, bout.reshape(1, 1))

    return jnp.concatenate([mlp(pre0), mlp(pre1)], axis=0)
